# baseline (speedup 1.0000x reference)
import functools

import jax
import jax.numpy as jnp
from jax import lax
from jax.experimental import pallas as pl
from jax.experimental.pallas import tpu as pltpu
from jax.experimental.pallas import tpu_sc as plsc

EPS = 1e-06
BIG = 1e9
L = 50
B = 1024
NLANE = 16
NWORK = 32
NCHUNK = B // NLANE
CPW = NCHUNK // NWORK
CHW = L * NLANE
FCW = 8 * L * NLANE


def _sc_feat_body(idsA_hbm, idsB_hbm, tA_hbm, tB_hbm, othA_hbm, othB_hbm,
                  curt_hbm, fA_hbm, fB_hbm,
                  idsA_v, idsB_v, tA_v, tB_v,
                  othA_v, othB_v, curt_v,
                  acc_v, srA_v, srB_v, fA_v, fB_v, cpw):
    wid = lax.axis_index("s") * 2 + lax.axis_index("c")

    def row(ref, i):
        return ref[pl.ds(i * NLANE, NLANE)]

    for ci in range(cpw):
        chunk = wid * cpw + ci
        pltpu.sync_copy(idsA_hbm.at[pl.ds(chunk * CHW, CHW)], idsA_v)
        pltpu.sync_copy(idsB_hbm.at[pl.ds(chunk * CHW, CHW)], idsB_v)
        pltpu.sync_copy(tA_hbm.at[pl.ds(chunk * CHW, CHW)], tA_v)
        pltpu.sync_copy(tB_hbm.at[pl.ds(chunk * CHW, CHW)], tB_v)
        pltpu.sync_copy(othA_hbm.at[pl.ds(chunk * NLANE, NLANE)], othA_v)
        pltpu.sync_copy(othB_hbm.at[pl.ds(chunk * NLANE, NLANE)], othB_v)
        pltpu.sync_copy(curt_hbm.at[pl.ds(chunk * NLANE, NLANE)], curt_v)

        def pass1_i(i, _):
            ai = row(idsA_v, i)
            bi = row(idsB_v, i)
            ta_i = row(tA_v, i)
            tb_i = row(tB_v, i)
            zi = jnp.zeros((NLANE,), jnp.int32)
            zf = jnp.zeros((NLANE,), jnp.float32)
            bigv = jnp.full((NLANE,), BIG, jnp.float32)

            def jbody(j, carry, lower):
                (cAA, srA, mnAA, mxAA, cAB, mnAB, mxAB, lastAB,
                 cBB, srB, mnBB, mxBB, cBA, mnBA, mxBA, lastBA) = carry
                aj = row(idsA_v, j)
                bj = row(idsB_v, j)
                taj = row(tA_v, j)
                tbj = row(tB_v, j)
                mAA = aj == ai
                mAB = bj == ai
                mBB = bj == bi
                mBA = aj == bi
                one = jnp.int32(1)
                cAA = jnp.where(mAA, cAA + one, cAA)
                mnAA = jnp.where(mAA, jnp.minimum(mnAA, taj), mnAA)
                mxAA = jnp.where(mAA, jnp.maximum(mxAA, taj), mxAA)
                ltA = (taj <= ta_i) if lower else (taj < ta_i)
                srA = jnp.where(mAA & ltA, srA + one, srA)
                cAB = jnp.where(mAB, cAB + one, cAB)
                mnAB = jnp.where(mAB, jnp.minimum(mnAB, tbj), mnAB)
                mxAB = jnp.where(mAB, jnp.maximum(mxAB, tbj), mxAB)
                lastAB = jnp.where(mAB & (bj != 0), tbj, lastAB)
                cBB = jnp.where(mBB, cBB + one, cBB)
                mnBB = jnp.where(mBB, jnp.minimum(mnBB, tbj), mnBB)
                mxBB = jnp.where(mBB, jnp.maximum(mxBB, tbj), mxBB)
                ltB = (tbj <= tb_i) if lower else (tbj < tb_i)
                srB = jnp.where(mBB & ltB, srB + one, srB)
                cBA = jnp.where(mBA, cBA + one, cBA)
                mnBA = jnp.where(mBA, jnp.minimum(mnBA, taj), mnBA)
                mxBA = jnp.where(mBA, jnp.maximum(mxBA, taj), mxBA)
                lastBA = jnp.where(mBA & (aj != 0), taj, lastBA)
                return (cAA, srA, mnAA, mxAA, cAB, mnAB, mxAB, lastAB,
                        cBB, srB, mnBB, mxBB, cBA, mnBA, mxBA, lastBA)

            init = (zi, zi, bigv, -bigv, zi, bigv, -bigv, zf,
                    zi, zi, bigv, -bigv, zi, bigv, -bigv, zf)
            carry = lax.fori_loop(
                0, i, functools.partial(jbody, lower=True), init)
            (cAA, srA, mnAA, mxAA, cAB, mnAB, mxAB, lastAB,
             cBB, srB, mnBB, mxBB, cBA, mnBA, mxBA, lastBA) = lax.fori_loop(
                i, L, functools.partial(jbody, lower=False), carry)

            fi = jnp.float32
            acc_v[pl.ds((0 * L + i) * NLANE, NLANE)] = cAA.astype(fi)
            acc_v[pl.ds((1 * L + i) * NLANE, NLANE)] = cAB.astype(fi)
            acc_v[pl.ds((2 * L + i) * NLANE, NLANE)] = cBB.astype(fi)
            acc_v[pl.ds((3 * L + i) * NLANE, NLANE)] = cBA.astype(fi)
            acc_v[pl.ds((4 * L + i) * NLANE, NLANE)] = mnAA
            acc_v[pl.ds((5 * L + i) * NLANE, NLANE)] = mxAA
            acc_v[pl.ds((6 * L + i) * NLANE, NLANE)] = mnAB
            acc_v[pl.ds((7 * L + i) * NLANE, NLANE)] = mxAB
            acc_v[pl.ds((8 * L + i) * NLANE, NLANE)] = lastAB
            acc_v[pl.ds((9 * L + i) * NLANE, NLANE)] = mnBB
            acc_v[pl.ds((10 * L + i) * NLANE, NLANE)] = mxBB
            acc_v[pl.ds((11 * L + i) * NLANE, NLANE)] = mnBA
            acc_v[pl.ds((12 * L + i) * NLANE, NLANE)] = mxBA
            acc_v[pl.ds((13 * L + i) * NLANE, NLANE)] = lastBA
            srA_v[pl.ds(i * NLANE, NLANE)] = srA
            srB_v[pl.ds(i * NLANE, NLANE)] = srB
            return 0

        lax.fori_loop(0, L, pass1_i, 0)

        curt = curt_v[...]
        othA = othA_v[...]
        othB = othB_v[...]

        def pass2_i(i, _):
            ai = row(idsA_v, i)
            bi = row(idsB_v, i)
            cAA = row(acc_v, 0 * L + i)
            cAB = row(acc_v, 1 * L + i)
            cBB = row(acc_v, 2 * L + i)
            cBA = row(acc_v, 3 * L + i)
            ispAA = jax.lax.shift_right_logical(cAA.astype(jnp.int32), 1)
            ispAB = jax.lax.shift_right_logical(cAB.astype(jnp.int32), 1)
            ispBB = jax.lax.shift_right_logical(cBB.astype(jnp.int32), 1)
            ispBA = jax.lax.shift_right_logical(cBA.astype(jnp.int32), 1)
            spAA = ispAA.astype(jnp.float32)
            spAB = ispAB.astype(jnp.float32)
            spBB = ispBB.astype(jnp.float32)
            spBA = ispBA.astype(jnp.float32)
            zf = jnp.zeros((NLANE,), jnp.float32)

            def jbody2(j, carry):
                vspAA, vspAB, vspBB, vspBA = carry
                aj = row(idsA_v, j)
                bj = row(idsB_v, j)
                taj = row(tA_v, j)
                tbj = row(tB_v, j)
                srAj = row(srA_v, j)
                srBj = row(srB_v, j)
                vspAA = jnp.where((aj == ai) & (srAj == ispAA), taj, vspAA)
                vspAB = jnp.where((bj == ai) & (srBj == ispAB), tbj, vspAB)
                vspBB = jnp.where((bj == bi) & (srBj == ispBB), tbj, vspBB)
                vspBA = jnp.where((aj == bi) & (srAj == ispBA), taj, vspBA)
                return vspAA, vspAB, vspBB, vspBA

            vspAA, vspAB, vspBB, vspBA = lax.fori_loop(
                0, L, jbody2, (zf, zf, zf, zf))

            def side(ids_i, t_i, oth, c_s, mn_s, mx_s, vsp_s, sp_s,
                     c_o, mn_o, mx_o, vsp_o, sp_o, last_o, f_v):
                keymask = ids_i != 0
                is_other = jnp.where(ids_i == oth, 1.0, 0.0)
                connects = jnp.where(c_o > 0.5, 1.0, 0.0)
                freq_asym = jnp.where(c_o > 0.5, c_s / (c_o + EPS), 0.0)
                rec_self = curt - t_i
                rec_other = curt - last_o
                temp_asym = jnp.where(rec_self > EPS,
                                      rec_other / (rec_self + EPS), 0.0)
                iat_self = jnp.where(
                    (c_s > 1.5) & keymask,
                    (mx_s - mn_s) / jnp.maximum(c_s - 1.0, 1.0), 0.0)
                iat_other = jnp.where(
                    (c_o > 1.5) & keymask,
                    (mx_o - mn_o) / jnp.maximum(c_o - 1.0, 1.0), 0.0)
                iat_asym = jnp.where(iat_other > EPS,
                                     iat_self / (iat_other + EPS), 0.0)
                r_self = jnp.where(
                    (c_s > 3.5) & keymask,
                    (mx_s - vsp_s) / jnp.maximum(c_s - sp_s - 1.0, 1.0), 0.0)
                r_other = jnp.where(
                    (c_o > 3.5) & keymask,
                    (mx_o - vsp_o) / jnp.maximum(c_o - sp_o - 1.0, 1.0), 0.0)
                r_asym = jnp.where(r_other > EPS,
                                   r_self / (r_other + EPS), 0.0)
                f_v[pl.ds((0 * L + i) * NLANE, NLANE)] = c_s
                f_v[pl.ds((1 * L + i) * NLANE, NLANE)] = c_o
                f_v[pl.ds((2 * L + i) * NLANE, NLANE)] = is_other
                f_v[pl.ds((3 * L + i) * NLANE, NLANE)] = connects
                f_v[pl.ds((4 * L + i) * NLANE, NLANE)] = freq_asym
                f_v[pl.ds((5 * L + i) * NLANE, NLANE)] = temp_asym
                f_v[pl.ds((6 * L + i) * NLANE, NLANE)] = iat_asym
                f_v[pl.ds((7 * L + i) * NLANE, NLANE)] = r_asym

            side(ai, row(tA_v, i), othA, cAA, row(acc_v, 4 * L + i),
                 row(acc_v, 5 * L + i), vspAA, spAA, cAB,
                 row(acc_v, 6 * L + i), row(acc_v, 7 * L + i), vspAB, spAB,
                 row(acc_v, 8 * L + i), fA_v)
            side(bi, row(tB_v, i), othB, cBB, row(acc_v, 9 * L + i),
                 row(acc_v, 10 * L + i), vspBB, spBB, cBA,
                 row(acc_v, 11 * L + i), row(acc_v, 12 * L + i), vspBA, spBA,
                 row(acc_v, 13 * L + i), fB_v)
            return 0

        lax.fori_loop(0, L, pass2_i, 0)

        pltpu.sync_copy(fA_v, fA_hbm.at[pl.ds(chunk * FCW, FCW)])
        pltpu.sync_copy(fB_v, fB_hbm.at[pl.ds(chunk * FCW, FCW)])


def _sc_features(idsA_f, idsB_f, tA_f, tB_f, othA, othB, curt):
    nchunk = idsA_f.shape[0] // CHW
    cpw = nchunk // NWORK
    mesh = plsc.VectorSubcoreMesh(core_axis_name="c", subcore_axis_name="s")
    feat_ty = jax.ShapeDtypeStruct((nchunk * FCW,), jnp.float32)
    kfn = functools.partial(
        pl.kernel, mesh=mesh,
        out_type=[feat_ty, feat_ty],
        scratch_types=[
            pltpu.VMEM((CHW,), jnp.int32),
            pltpu.VMEM((CHW,), jnp.int32),
            pltpu.VMEM((CHW,), jnp.float32),
            pltpu.VMEM((CHW,), jnp.float32),
            pltpu.VMEM((NLANE,), jnp.int32),
            pltpu.VMEM((NLANE,), jnp.int32),
            pltpu.VMEM((NLANE,), jnp.float32),
            pltpu.VMEM((14 * CHW,), jnp.float32),
            pltpu.VMEM((CHW,), jnp.int32),
            pltpu.VMEM((CHW,), jnp.int32),
            pltpu.VMEM((FCW,), jnp.float32),
            pltpu.VMEM((FCW,), jnp.float32),
        ],
    )(functools.partial(_sc_feat_body, cpw=cpw))
    return kfn(idsA_f, idsB_f, tA_f, tB_f, othA, othB, curt)


def _encode_body(fA_ref, fB_ref, w1_ref, b1_ref, w2_ref, b2_ref,
                 outA_ref, outB_ref):
    Bb = fA_ref.shape[2]
    w1 = w1_ref[...]
    b1 = b1_ref[...]
    w2 = w2_ref[...]
    b2 = b2_ref[...]

    def one_side(f_ref, out_ref):
        for l2 in range(L // 2):
            cols = []
            for l in (2 * l2, 2 * l2 + 1):
                g = jnp.zeros((64, Bb), jnp.float32)
                for fi in range(8):
                    row = f_ref[fi, pl.ds(l, 1), :]
                    g = g + jnp.maximum(w1 * row + b1, 0.0)
                cols.append(g)
            g2 = jnp.concatenate(cols, axis=1)
            o = lax.dot_general(g2, w2, (((0,), (0,)), ((), ())),
                                preferred_element_type=jnp.float32)
            o = o + 8.0 * b2
            out_ref[:, pl.ds(2 * l2 * 64, 128)] = jnp.concatenate(
                [o[:Bb, :], o[Bb:, :]], axis=1)

    one_side(fA_ref, outA_ref)
    one_side(fB_ref, outB_ref)


def _chunk_major(x):
    return x.reshape(NCHUNK, NLANE, L).transpose(0, 2, 1).reshape(-1)


def _encode(fA, fB, w1c, b1c, w2, b2r):
    bh = fA.shape[2]
    EBB = 128
    out_shape = jax.ShapeDtypeStruct((bh, L * 64), jnp.float32)
    return pl.pallas_call(
        _encode_body,
        grid=(bh // EBB,),
        in_specs=[
            pl.BlockSpec((8, L, EBB), lambda i: (0, 0, i)),
            pl.BlockSpec((8, L, EBB), lambda i: (0, 0, i)),
            pl.BlockSpec((64, 1), lambda i: (0, 0)),
            pl.BlockSpec((64, 1), lambda i: (0, 0)),
            pl.BlockSpec((64, 64), lambda i: (0, 0)),
            pl.BlockSpec((1, 64), lambda i: (0, 0)),
        ],
        out_specs=[
            pl.BlockSpec((EBB, L * 64), lambda i: (i, 0)),
            pl.BlockSpec((EBB, L * 64), lambda i: (i, 0)),
        ],
        out_shape=[out_shape, out_shape],
    )(fA, fB, w1c, b1c, w2, b2r)


@jax.jit
def _run(idsA, idsB, tA, tB, othA, othB, curt, w1c, b1c, w2, b2r):
    BH = B // 2
    NCH = BH // NLANE
    idsA_f = _chunk_major(idsA)
    idsB_f = _chunk_major(idsB)
    tA_f = _chunk_major(tA)
    tB_f = _chunk_major(tB)
    halves = []
    W = NCH * CHW
    feats = []
    for h in range(2):
        feats.append(_sc_features(
            lax.dynamic_slice_in_dim(idsA_f, h * W, W),
            lax.dynamic_slice_in_dim(idsB_f, h * W, W),
            lax.dynamic_slice_in_dim(tA_f, h * W, W),
            lax.dynamic_slice_in_dim(tB_f, h * W, W),
            lax.dynamic_slice_in_dim(othA, h * BH, BH),
            lax.dynamic_slice_in_dim(othB, h * BH, BH),
            lax.dynamic_slice_in_dim(curt, h * BH, BH)))
    for h in range(2):
        fA_f, fB_f = feats[h]
        fA = fA_f.reshape(NCH, 8, L, NLANE).transpose(1, 2, 0, 3).reshape(
            8, L, BH)
        fB = fB_f.reshape(NCH, 8, L, NLANE).transpose(1, 2, 0, 3).reshape(
            8, L, BH)
        halves.append(_encode(fA, fB, w1c, b1c, w2, b2r))
    outA = jnp.concatenate([halves[0][0], halves[1][0]], axis=0)
    outB = jnp.concatenate([halves[0][1], halves[1][1]], axis=0)
    return outA.reshape(B, L, 64), outB.reshape(B, L, 64)


def kernel(src_padded_nodes_neighbor_ids, dst_padded_nodes_neighbor_ids,
           src_node_ids, dst_node_ids, node_interact_times,
           src_padded_nodes_neighbor_times, dst_padded_nodes_neighbor_times,
           W1, b1, W2, b2):
    idsA = src_padded_nodes_neighbor_ids.astype(jnp.int32)
    idsB = dst_padded_nodes_neighbor_ids.astype(jnp.int32)
    othA = dst_node_ids.astype(jnp.int32)
    othB = src_node_ids.astype(jnp.int32)
    w1c = W1.reshape(1, 64).T
    b1c = b1.reshape(64, 1)
    b2r = b2.reshape(1, 64)
    return _run(idsA, idsB, src_padded_nodes_neighbor_times,
                dst_padded_nodes_neighbor_times, othA, othB,
                node_interact_times, w1c, b1c, W2, b2r)


# speedup vs baseline: 1.0835x; 1.0835x over previous
import functools

import jax
import jax.numpy as jnp
from jax import lax
from jax.experimental import pallas as pl
from jax.experimental.pallas import tpu as pltpu
from jax.experimental.pallas import tpu_sc as plsc

EPS = 1e-06
BIG = 1e9
L = 50
B = 1024
NLANE = 16
NWORK = 32
NCHUNK = B // NLANE
CPW = NCHUNK // NWORK
CHW = L * NLANE
FCW = 8 * L * NLANE


def _sc_feat_body(idsA_hbm, idsB_hbm, tA_hbm, tB_hbm, othA_hbm, othB_hbm,
                  curt_hbm, fA_hbm, fB_hbm,
                  idsA_v, idsB_v, tA_v, tB_v,
                  othA_v, othB_v, curt_v,
                  acc_v, srA_v, srB_v, fA_v, fB_v, cpw):
    wid = lax.axis_index("s") * 2 + lax.axis_index("c")

    def row(ref, i):
        return ref[pl.ds(i * NLANE, NLANE)]

    for ci in range(cpw):
        chunk = wid * cpw + ci
        pltpu.sync_copy(idsA_hbm.at[pl.ds(chunk * CHW, CHW)], idsA_v)
        pltpu.sync_copy(idsB_hbm.at[pl.ds(chunk * CHW, CHW)], idsB_v)
        pltpu.sync_copy(tA_hbm.at[pl.ds(chunk * CHW, CHW)], tA_v)
        pltpu.sync_copy(tB_hbm.at[pl.ds(chunk * CHW, CHW)], tB_v)
        pltpu.sync_copy(othA_hbm.at[pl.ds(chunk * NLANE, NLANE)], othA_v)
        pltpu.sync_copy(othB_hbm.at[pl.ds(chunk * NLANE, NLANE)], othB_v)
        pltpu.sync_copy(curt_hbm.at[pl.ds(chunk * NLANE, NLANE)], curt_v)

        def pass1_i(i, _):
            ai = row(idsA_v, i)
            bi = row(idsB_v, i)
            ta_i = row(tA_v, i)
            tb_i = row(tB_v, i)
            zi = jnp.zeros((NLANE,), jnp.int32)
            zf = jnp.zeros((NLANE,), jnp.float32)
            bigv = jnp.full((NLANE,), BIG, jnp.float32)

            def jbody(j, carry, lower):
                (cAA, srA, mnAA, mxAA, cAB, mnAB, mxAB, lastAB,
                 cBB, srB, mnBB, mxBB, cBA, mnBA, mxBA, lastBA) = carry
                aj = row(idsA_v, j)
                bj = row(idsB_v, j)
                taj = row(tA_v, j)
                tbj = row(tB_v, j)
                mAA = aj == ai
                mAB = bj == ai
                mBB = bj == bi
                mBA = aj == bi
                one = jnp.int32(1)
                cAA = jnp.where(mAA, cAA + one, cAA)
                mnAA = jnp.where(mAA, jnp.minimum(mnAA, taj), mnAA)
                mxAA = jnp.where(mAA, jnp.maximum(mxAA, taj), mxAA)
                ltA = (taj <= ta_i) if lower else (taj < ta_i)
                srA = jnp.where(mAA & ltA, srA + one, srA)
                cAB = jnp.where(mAB, cAB + one, cAB)
                mnAB = jnp.where(mAB, jnp.minimum(mnAB, tbj), mnAB)
                mxAB = jnp.where(mAB, jnp.maximum(mxAB, tbj), mxAB)
                lastAB = jnp.where(mAB & (bj != 0), tbj, lastAB)
                cBB = jnp.where(mBB, cBB + one, cBB)
                mnBB = jnp.where(mBB, jnp.minimum(mnBB, tbj), mnBB)
                mxBB = jnp.where(mBB, jnp.maximum(mxBB, tbj), mxBB)
                ltB = (tbj <= tb_i) if lower else (tbj < tb_i)
                srB = jnp.where(mBB & ltB, srB + one, srB)
                cBA = jnp.where(mBA, cBA + one, cBA)
                mnBA = jnp.where(mBA, jnp.minimum(mnBA, taj), mnBA)
                mxBA = jnp.where(mBA, jnp.maximum(mxBA, taj), mxBA)
                lastBA = jnp.where(mBA & (aj != 0), taj, lastBA)
                return (cAA, srA, mnAA, mxAA, cAB, mnAB, mxAB, lastAB,
                        cBB, srB, mnBB, mxBB, cBA, mnBA, mxBA, lastBA)

            init = (zi, zi, bigv, -bigv, zi, bigv, -bigv, zf,
                    zi, zi, bigv, -bigv, zi, bigv, -bigv, zf)
            carry = lax.fori_loop(
                0, i, functools.partial(jbody, lower=True), init)
            (cAA, srA, mnAA, mxAA, cAB, mnAB, mxAB, lastAB,
             cBB, srB, mnBB, mxBB, cBA, mnBA, mxBA, lastBA) = lax.fori_loop(
                i, L, functools.partial(jbody, lower=False), carry)

            fi = jnp.float32
            acc_v[pl.ds((0 * L + i) * NLANE, NLANE)] = cAA.astype(fi)
            acc_v[pl.ds((1 * L + i) * NLANE, NLANE)] = cAB.astype(fi)
            acc_v[pl.ds((2 * L + i) * NLANE, NLANE)] = cBB.astype(fi)
            acc_v[pl.ds((3 * L + i) * NLANE, NLANE)] = cBA.astype(fi)
            acc_v[pl.ds((4 * L + i) * NLANE, NLANE)] = mnAA
            acc_v[pl.ds((5 * L + i) * NLANE, NLANE)] = mxAA
            acc_v[pl.ds((6 * L + i) * NLANE, NLANE)] = mnAB
            acc_v[pl.ds((7 * L + i) * NLANE, NLANE)] = mxAB
            acc_v[pl.ds((8 * L + i) * NLANE, NLANE)] = lastAB
            acc_v[pl.ds((9 * L + i) * NLANE, NLANE)] = mnBB
            acc_v[pl.ds((10 * L + i) * NLANE, NLANE)] = mxBB
            acc_v[pl.ds((11 * L + i) * NLANE, NLANE)] = mnBA
            acc_v[pl.ds((12 * L + i) * NLANE, NLANE)] = mxBA
            acc_v[pl.ds((13 * L + i) * NLANE, NLANE)] = lastBA
            srA_v[pl.ds(i * NLANE, NLANE)] = srA
            srB_v[pl.ds(i * NLANE, NLANE)] = srB
            return 0

        lax.fori_loop(0, L, pass1_i, 0)

        curt = curt_v[...]
        othA = othA_v[...]
        othB = othB_v[...]

        def pass2_i(i, _):
            ai = row(idsA_v, i)
            bi = row(idsB_v, i)
            cAA = row(acc_v, 0 * L + i)
            cAB = row(acc_v, 1 * L + i)
            cBB = row(acc_v, 2 * L + i)
            cBA = row(acc_v, 3 * L + i)
            ispAA = jax.lax.shift_right_logical(cAA.astype(jnp.int32), 1)
            ispAB = jax.lax.shift_right_logical(cAB.astype(jnp.int32), 1)
            ispBB = jax.lax.shift_right_logical(cBB.astype(jnp.int32), 1)
            ispBA = jax.lax.shift_right_logical(cBA.astype(jnp.int32), 1)
            spAA = ispAA.astype(jnp.float32)
            spAB = ispAB.astype(jnp.float32)
            spBB = ispBB.astype(jnp.float32)
            spBA = ispBA.astype(jnp.float32)
            zf = jnp.zeros((NLANE,), jnp.float32)

            def jbody2(j, carry):
                vspAA, vspAB, vspBB, vspBA = carry
                aj = row(idsA_v, j)
                bj = row(idsB_v, j)
                taj = row(tA_v, j)
                tbj = row(tB_v, j)
                srAj = row(srA_v, j)
                srBj = row(srB_v, j)
                vspAA = jnp.where((aj == ai) & (srAj == ispAA), taj, vspAA)
                vspAB = jnp.where((bj == ai) & (srBj == ispAB), tbj, vspAB)
                vspBB = jnp.where((bj == bi) & (srBj == ispBB), tbj, vspBB)
                vspBA = jnp.where((aj == bi) & (srAj == ispBA), taj, vspBA)
                return vspAA, vspAB, vspBB, vspBA

            vspAA, vspAB, vspBB, vspBA = lax.fori_loop(
                0, L, jbody2, (zf, zf, zf, zf))

            def side(ids_i, t_i, oth, c_s, mn_s, mx_s, vsp_s, sp_s,
                     c_o, mn_o, mx_o, vsp_o, sp_o, last_o, f_v):
                keymask = ids_i != 0
                is_other = jnp.where(ids_i == oth, 1.0, 0.0)
                connects = jnp.where(c_o > 0.5, 1.0, 0.0)
                freq_asym = jnp.where(c_o > 0.5, c_s / (c_o + EPS), 0.0)
                rec_self = curt - t_i
                rec_other = curt - last_o
                temp_asym = jnp.where(rec_self > EPS,
                                      rec_other / (rec_self + EPS), 0.0)
                iat_self = jnp.where(
                    (c_s > 1.5) & keymask,
                    (mx_s - mn_s) / jnp.maximum(c_s - 1.0, 1.0), 0.0)
                iat_other = jnp.where(
                    (c_o > 1.5) & keymask,
                    (mx_o - mn_o) / jnp.maximum(c_o - 1.0, 1.0), 0.0)
                iat_asym = jnp.where(iat_other > EPS,
                                     iat_self / (iat_other + EPS), 0.0)
                r_self = jnp.where(
                    (c_s > 3.5) & keymask,
                    (mx_s - vsp_s) / jnp.maximum(c_s - sp_s - 1.0, 1.0), 0.0)
                r_other = jnp.where(
                    (c_o > 3.5) & keymask,
                    (mx_o - vsp_o) / jnp.maximum(c_o - sp_o - 1.0, 1.0), 0.0)
                r_asym = jnp.where(r_other > EPS,
                                   r_self / (r_other + EPS), 0.0)
                f_v[pl.ds((0 * L + i) * NLANE, NLANE)] = c_s
                f_v[pl.ds((1 * L + i) * NLANE, NLANE)] = c_o
                f_v[pl.ds((2 * L + i) * NLANE, NLANE)] = is_other
                f_v[pl.ds((3 * L + i) * NLANE, NLANE)] = connects
                f_v[pl.ds((4 * L + i) * NLANE, NLANE)] = freq_asym
                f_v[pl.ds((5 * L + i) * NLANE, NLANE)] = temp_asym
                f_v[pl.ds((6 * L + i) * NLANE, NLANE)] = iat_asym
                f_v[pl.ds((7 * L + i) * NLANE, NLANE)] = r_asym

            side(ai, row(tA_v, i), othA, cAA, row(acc_v, 4 * L + i),
                 row(acc_v, 5 * L + i), vspAA, spAA, cAB,
                 row(acc_v, 6 * L + i), row(acc_v, 7 * L + i), vspAB, spAB,
                 row(acc_v, 8 * L + i), fA_v)
            side(bi, row(tB_v, i), othB, cBB, row(acc_v, 9 * L + i),
                 row(acc_v, 10 * L + i), vspBB, spBB, cBA,
                 row(acc_v, 11 * L + i), row(acc_v, 12 * L + i), vspBA, spBA,
                 row(acc_v, 13 * L + i), fB_v)
            return 0

        lax.fori_loop(0, L, pass2_i, 0)

        pltpu.sync_copy(fA_v, fA_hbm.at[pl.ds(chunk * FCW, FCW)])
        pltpu.sync_copy(fB_v, fB_hbm.at[pl.ds(chunk * FCW, FCW)])


def _sc_features(idsA_f, idsB_f, tA_f, tB_f, othA, othB, curt):
    nchunk = idsA_f.shape[0] // CHW
    cpw = nchunk // NWORK
    mesh = plsc.VectorSubcoreMesh(core_axis_name="c", subcore_axis_name="s")
    feat_ty = jax.ShapeDtypeStruct((nchunk * FCW,), jnp.float32)
    kfn = functools.partial(
        pl.kernel, mesh=mesh,
        out_type=[feat_ty, feat_ty],
        scratch_types=[
            pltpu.VMEM((CHW,), jnp.int32),
            pltpu.VMEM((CHW,), jnp.int32),
            pltpu.VMEM((CHW,), jnp.float32),
            pltpu.VMEM((CHW,), jnp.float32),
            pltpu.VMEM((NLANE,), jnp.int32),
            pltpu.VMEM((NLANE,), jnp.int32),
            pltpu.VMEM((NLANE,), jnp.float32),
            pltpu.VMEM((14 * CHW,), jnp.float32),
            pltpu.VMEM((CHW,), jnp.int32),
            pltpu.VMEM((CHW,), jnp.int32),
            pltpu.VMEM((FCW,), jnp.float32),
            pltpu.VMEM((FCW,), jnp.float32),
        ],
    )(functools.partial(_sc_feat_body, cpw=cpw))
    return kfn(idsA_f, idsB_f, tA_f, tB_f, othA, othB, curt)


def _encode_body(fA_ref, fB_ref, w1_ref, b1_ref, w2_ref, b2_ref,
                 outA_ref, outB_ref):
    Bb = fA_ref.shape[2]
    w1 = w1_ref[...]
    b1 = b1_ref[...]
    w2 = w2_ref[...]
    b2 = b2_ref[...]

    def one_side(f_ref, out_ref):
        for l2 in range(L // 2):
            cols = []
            for l in (2 * l2, 2 * l2 + 1):
                g = jnp.zeros((64, Bb), jnp.float32)
                for fi in range(8):
                    row = f_ref[fi, pl.ds(l, 1), :]
                    g = g + jnp.maximum(w1 * row + b1, 0.0)
                cols.append(g)
            g2 = jnp.concatenate(cols, axis=1)
            o = lax.dot_general(g2, w2, (((0,), (0,)), ((), ())),
                                preferred_element_type=jnp.float32)
            o = o + 8.0 * b2
            out_ref[:, pl.ds(2 * l2 * 64, 128)] = jnp.concatenate(
                [o[:Bb, :], o[Bb:, :]], axis=1)

    one_side(fA_ref, outA_ref)
    one_side(fB_ref, outB_ref)


def _chunk_major(x):
    return x.reshape(NCHUNK, NLANE, L).transpose(0, 2, 1).reshape(-1)


def _encode(fA, fB, w1c, b1c, w2, b2r):
    bh = fA.shape[2]
    EBB = 128
    out_shape = jax.ShapeDtypeStruct((bh, L * 64), jnp.float32)
    return pl.pallas_call(
        _encode_body,
        grid=(bh // EBB,),
        in_specs=[
            pl.BlockSpec((8, L, EBB), lambda i: (0, 0, i)),
            pl.BlockSpec((8, L, EBB), lambda i: (0, 0, i)),
            pl.BlockSpec((64, 1), lambda i: (0, 0)),
            pl.BlockSpec((64, 1), lambda i: (0, 0)),
            pl.BlockSpec((64, 64), lambda i: (0, 0)),
            pl.BlockSpec((1, 64), lambda i: (0, 0)),
        ],
        out_specs=[
            pl.BlockSpec((EBB, L * 64), lambda i: (i, 0)),
            pl.BlockSpec((EBB, L * 64), lambda i: (i, 0)),
        ],
        out_shape=[out_shape, out_shape],
    )(fA, fB, w1c, b1c, w2, b2r)


@jax.jit
def _run(idsA, idsB, tA, tB, othA, othB, curt, w1c, b1c, w2, b2r):
    fA_f, fB_f = _sc_features(_chunk_major(idsA), _chunk_major(idsB),
                              _chunk_major(tA), _chunk_major(tB),
                              othA, othB, curt)
    fA = fA_f.reshape(NCHUNK, 8, L, NLANE).transpose(1, 2, 0, 3).reshape(
        8, L, B)
    fB = fB_f.reshape(NCHUNK, 8, L, NLANE).transpose(1, 2, 0, 3).reshape(
        8, L, B)
    outA, outB = _encode(fA, fB, w1c, b1c, w2, b2r)
    return outA.reshape(B, L, 64), outB.reshape(B, L, 64)


def kernel(src_padded_nodes_neighbor_ids, dst_padded_nodes_neighbor_ids,
           src_node_ids, dst_node_ids, node_interact_times,
           src_padded_nodes_neighbor_times, dst_padded_nodes_neighbor_times,
           W1, b1, W2, b2):
    idsA = src_padded_nodes_neighbor_ids.astype(jnp.int32)
    idsB = dst_padded_nodes_neighbor_ids.astype(jnp.int32)
    othA = dst_node_ids.astype(jnp.int32)
    othB = src_node_ids.astype(jnp.int32)
    w1c = W1.reshape(1, 64).T
    b1c = b1.reshape(64, 1)
    b2r = b2.reshape(1, 64)
    return _run(idsA, idsB, src_padded_nodes_neighbor_times,
                dst_padded_nodes_neighbor_times, othA, othB,
                node_interact_times, w1c, b1c, W2, b2r)


# speedup vs baseline: 1.2132x; 1.1198x over previous
import functools

import jax
import jax.numpy as jnp
from jax import lax
from jax.experimental import pallas as pl
from jax.experimental.pallas import tpu as pltpu
from jax.experimental.pallas import tpu_sc as plsc

EPS = 1e-06
BIG = 1e9
L = 50
B = 1024
NLANE = 16
NWORK = 32
NCHUNK = B // NLANE
CPW = NCHUNK // NWORK
CHW = L * NLANE
FCW = 8 * L * NLANE


def _sc_feat_body(idsA_hbm, idsB_hbm, tA_hbm, tB_hbm, othA_hbm, othB_hbm,
                  curt_hbm, fA_hbm, fB_hbm,
                  idsA_v, idsB_v, tA_v, tB_v,
                  othA_v, othB_v, curt_v,
                  acc_v, srA_v, srB_v, fA_v, fB_v, cpw):
    wid = lax.axis_index("s") * 2 + lax.axis_index("c")

    def row(ref, i):
        return ref[pl.ds(i * NLANE, NLANE)]

    for ci in range(cpw):
        chunk = wid * cpw + ci
        pltpu.sync_copy(idsA_hbm.at[pl.ds(chunk * CHW, CHW)], idsA_v)
        pltpu.sync_copy(idsB_hbm.at[pl.ds(chunk * CHW, CHW)], idsB_v)
        pltpu.sync_copy(tA_hbm.at[pl.ds(chunk * CHW, CHW)], tA_v)
        pltpu.sync_copy(tB_hbm.at[pl.ds(chunk * CHW, CHW)], tB_v)
        pltpu.sync_copy(othA_hbm.at[pl.ds(chunk * NLANE, NLANE)], othA_v)
        pltpu.sync_copy(othB_hbm.at[pl.ds(chunk * NLANE, NLANE)], othB_v)
        pltpu.sync_copy(curt_hbm.at[pl.ds(chunk * NLANE, NLANE)], curt_v)

        def pass1_i(i, _):
            ai = row(idsA_v, i)
            bi = row(idsB_v, i)
            ta_i = row(tA_v, i)
            tb_i = row(tB_v, i)
            zi = jnp.zeros((NLANE,), jnp.int32)
            zf = jnp.zeros((NLANE,), jnp.float32)
            bigv = jnp.full((NLANE,), BIG, jnp.float32)

            def jbody(j, carry, lower):
                (cAA, srA, mnAA, mxAA, cAB, mnAB, mxAB, lastAB,
                 cBB, srB, mnBB, mxBB, cBA, mnBA, mxBA, lastBA) = carry
                aj = row(idsA_v, j)
                bj = row(idsB_v, j)
                taj = row(tA_v, j)
                tbj = row(tB_v, j)
                mAA = aj == ai
                mAB = bj == ai
                mBB = bj == bi
                mBA = aj == bi
                one = jnp.int32(1)
                cAA = jnp.where(mAA, cAA + one, cAA)
                mnAA = jnp.where(mAA, jnp.minimum(mnAA, taj), mnAA)
                mxAA = jnp.where(mAA, jnp.maximum(mxAA, taj), mxAA)
                ltA = (taj <= ta_i) if lower else (taj < ta_i)
                srA = jnp.where(mAA & ltA, srA + one, srA)
                cAB = jnp.where(mAB, cAB + one, cAB)
                mnAB = jnp.where(mAB, jnp.minimum(mnAB, tbj), mnAB)
                mxAB = jnp.where(mAB, jnp.maximum(mxAB, tbj), mxAB)
                lastAB = jnp.where(mAB & (bj != 0), tbj, lastAB)
                cBB = jnp.where(mBB, cBB + one, cBB)
                mnBB = jnp.where(mBB, jnp.minimum(mnBB, tbj), mnBB)
                mxBB = jnp.where(mBB, jnp.maximum(mxBB, tbj), mxBB)
                ltB = (tbj <= tb_i) if lower else (tbj < tb_i)
                srB = jnp.where(mBB & ltB, srB + one, srB)
                cBA = jnp.where(mBA, cBA + one, cBA)
                mnBA = jnp.where(mBA, jnp.minimum(mnBA, taj), mnBA)
                mxBA = jnp.where(mBA, jnp.maximum(mxBA, taj), mxBA)
                lastBA = jnp.where(mBA & (aj != 0), taj, lastBA)
                return (cAA, srA, mnAA, mxAA, cAB, mnAB, mxAB, lastAB,
                        cBB, srB, mnBB, mxBB, cBA, mnBA, mxBA, lastBA)

            init = (zi, zi, bigv, -bigv, zi, bigv, -bigv, zf,
                    zi, zi, bigv, -bigv, zi, bigv, -bigv, zf)
            carry = lax.fori_loop(
                0, i, functools.partial(jbody, lower=True), init)
            (cAA, srA, mnAA, mxAA, cAB, mnAB, mxAB, lastAB,
             cBB, srB, mnBB, mxBB, cBA, mnBA, mxBA, lastBA) = lax.fori_loop(
                i, L, functools.partial(jbody, lower=False), carry)

            fi = jnp.float32
            acc_v[pl.ds((0 * L + i) * NLANE, NLANE)] = cAA.astype(fi)
            acc_v[pl.ds((1 * L + i) * NLANE, NLANE)] = cAB.astype(fi)
            acc_v[pl.ds((2 * L + i) * NLANE, NLANE)] = cBB.astype(fi)
            acc_v[pl.ds((3 * L + i) * NLANE, NLANE)] = cBA.astype(fi)
            acc_v[pl.ds((4 * L + i) * NLANE, NLANE)] = mnAA
            acc_v[pl.ds((5 * L + i) * NLANE, NLANE)] = mxAA
            acc_v[pl.ds((6 * L + i) * NLANE, NLANE)] = mnAB
            acc_v[pl.ds((7 * L + i) * NLANE, NLANE)] = mxAB
            acc_v[pl.ds((8 * L + i) * NLANE, NLANE)] = lastAB
            acc_v[pl.ds((9 * L + i) * NLANE, NLANE)] = mnBB
            acc_v[pl.ds((10 * L + i) * NLANE, NLANE)] = mxBB
            acc_v[pl.ds((11 * L + i) * NLANE, NLANE)] = mnBA
            acc_v[pl.ds((12 * L + i) * NLANE, NLANE)] = mxBA
            acc_v[pl.ds((13 * L + i) * NLANE, NLANE)] = lastBA
            srA_v[pl.ds(i * NLANE, NLANE)] = srA
            srB_v[pl.ds(i * NLANE, NLANE)] = srB
            return 0

        lax.fori_loop(0, L, pass1_i, 0)

        curt = curt_v[...]
        othA = othA_v[...]
        othB = othB_v[...]

        def pass2_i(i, _):
            ai = row(idsA_v, i)
            bi = row(idsB_v, i)
            cAA = row(acc_v, 0 * L + i)
            cAB = row(acc_v, 1 * L + i)
            cBB = row(acc_v, 2 * L + i)
            cBA = row(acc_v, 3 * L + i)
            ispAA = jax.lax.shift_right_logical(cAA.astype(jnp.int32), 1)
            ispAB = jax.lax.shift_right_logical(cAB.astype(jnp.int32), 1)
            ispBB = jax.lax.shift_right_logical(cBB.astype(jnp.int32), 1)
            ispBA = jax.lax.shift_right_logical(cBA.astype(jnp.int32), 1)
            spAA = ispAA.astype(jnp.float32)
            spAB = ispAB.astype(jnp.float32)
            spBB = ispBB.astype(jnp.float32)
            spBA = ispBA.astype(jnp.float32)
            zf = jnp.zeros((NLANE,), jnp.float32)

            def jbody2(j, carry):
                vspAA, vspAB, vspBB, vspBA = carry
                aj = row(idsA_v, j)
                bj = row(idsB_v, j)
                taj = row(tA_v, j)
                tbj = row(tB_v, j)
                srAj = row(srA_v, j)
                srBj = row(srB_v, j)
                vspAA = jnp.where((aj == ai) & (srAj == ispAA), taj, vspAA)
                vspAB = jnp.where((bj == ai) & (srBj == ispAB), tbj, vspAB)
                vspBB = jnp.where((bj == bi) & (srBj == ispBB), tbj, vspBB)
                vspBA = jnp.where((aj == bi) & (srAj == ispBA), taj, vspBA)
                return vspAA, vspAB, vspBB, vspBA

            vspAA, vspAB, vspBB, vspBA = lax.fori_loop(
                0, L, jbody2, (zf, zf, zf, zf))

            def side(ids_i, t_i, oth, c_s, mn_s, mx_s, vsp_s, sp_s,
                     c_o, mn_o, mx_o, vsp_o, sp_o, last_o, f_v):
                keymask = ids_i != 0
                is_other = jnp.where(ids_i == oth, 1.0, 0.0)
                connects = jnp.where(c_o > 0.5, 1.0, 0.0)
                freq_asym = jnp.where(c_o > 0.5, c_s / (c_o + EPS), 0.0)
                rec_self = curt - t_i
                rec_other = curt - last_o
                temp_asym = jnp.where(rec_self > EPS,
                                      rec_other / (rec_self + EPS), 0.0)
                iat_self = jnp.where(
                    (c_s > 1.5) & keymask,
                    (mx_s - mn_s) / jnp.maximum(c_s - 1.0, 1.0), 0.0)
                iat_other = jnp.where(
                    (c_o > 1.5) & keymask,
                    (mx_o - mn_o) / jnp.maximum(c_o - 1.0, 1.0), 0.0)
                iat_asym = jnp.where(iat_other > EPS,
                                     iat_self / (iat_other + EPS), 0.0)
                r_self = jnp.where(
                    (c_s > 3.5) & keymask,
                    (mx_s - vsp_s) / jnp.maximum(c_s - sp_s - 1.0, 1.0), 0.0)
                r_other = jnp.where(
                    (c_o > 3.5) & keymask,
                    (mx_o - vsp_o) / jnp.maximum(c_o - sp_o - 1.0, 1.0), 0.0)
                r_asym = jnp.where(r_other > EPS,
                                   r_self / (r_other + EPS), 0.0)
                f_v[pl.ds((0 * L + i) * NLANE, NLANE)] = c_s
                f_v[pl.ds((1 * L + i) * NLANE, NLANE)] = c_o
                f_v[pl.ds((2 * L + i) * NLANE, NLANE)] = is_other
                f_v[pl.ds((3 * L + i) * NLANE, NLANE)] = connects
                f_v[pl.ds((4 * L + i) * NLANE, NLANE)] = freq_asym
                f_v[pl.ds((5 * L + i) * NLANE, NLANE)] = temp_asym
                f_v[pl.ds((6 * L + i) * NLANE, NLANE)] = iat_asym
                f_v[pl.ds((7 * L + i) * NLANE, NLANE)] = r_asym

            side(ai, row(tA_v, i), othA, cAA, row(acc_v, 4 * L + i),
                 row(acc_v, 5 * L + i), vspAA, spAA, cAB,
                 row(acc_v, 6 * L + i), row(acc_v, 7 * L + i), vspAB, spAB,
                 row(acc_v, 8 * L + i), fA_v)
            side(bi, row(tB_v, i), othB, cBB, row(acc_v, 9 * L + i),
                 row(acc_v, 10 * L + i), vspBB, spBB, cBA,
                 row(acc_v, 11 * L + i), row(acc_v, 12 * L + i), vspBA, spBA,
                 row(acc_v, 13 * L + i), fB_v)
            return 0

        lax.fori_loop(0, L, pass2_i, 0)

        pltpu.sync_copy(fA_v, fA_hbm.at[pl.ds(chunk * FCW, FCW)])
        pltpu.sync_copy(fB_v, fB_hbm.at[pl.ds(chunk * FCW, FCW)])


def _sc_features(idsA_f, idsB_f, tA_f, tB_f, othA, othB, curt):
    nchunk = idsA_f.shape[0] // CHW
    cpw = nchunk // NWORK
    mesh = plsc.VectorSubcoreMesh(core_axis_name="c", subcore_axis_name="s")
    feat_ty = jax.ShapeDtypeStruct((nchunk * FCW,), jnp.float32)
    kfn = functools.partial(
        pl.kernel, mesh=mesh,
        out_type=[feat_ty, feat_ty],
        scratch_types=[
            pltpu.VMEM((CHW,), jnp.int32),
            pltpu.VMEM((CHW,), jnp.int32),
            pltpu.VMEM((CHW,), jnp.float32),
            pltpu.VMEM((CHW,), jnp.float32),
            pltpu.VMEM((NLANE,), jnp.int32),
            pltpu.VMEM((NLANE,), jnp.int32),
            pltpu.VMEM((NLANE,), jnp.float32),
            pltpu.VMEM((14 * CHW,), jnp.float32),
            pltpu.VMEM((CHW,), jnp.int32),
            pltpu.VMEM((CHW,), jnp.int32),
            pltpu.VMEM((FCW,), jnp.float32),
            pltpu.VMEM((FCW,), jnp.float32),
        ],
    )(functools.partial(_sc_feat_body, cpw=cpw))
    return kfn(idsA_f, idsB_f, tA_f, tB_f, othA, othB, curt)


def _encode_body(fA_ref, fB_ref, w1_ref, b1_ref, w2_ref, b2_ref,
                 outA_ref, outB_ref):
    Bb = 128
    w1 = w1_ref[...]
    b1 = b1_ref[...]
    w2 = w2_ref[...]
    b2 = b2_ref[...]

    def one_side(f_ref, out_ref):
        x = f_ref[...]
        y = x.reshape(8, 8 * L, NLANE).swapaxes(0, 1).reshape(8 * L, Bb)
        for l2 in range(L // 2):
            cols = []
            for l in (2 * l2, 2 * l2 + 1):
                g = jnp.zeros((64, Bb), jnp.float32)
                for fi in range(8):
                    row = y[fi * L + l:fi * L + l + 1, :]
                    g = g + jnp.maximum(w1 * row + b1, 0.0)
                cols.append(g)
            g2 = jnp.concatenate(cols, axis=1)
            o = lax.dot_general(g2, w2, (((0,), (0,)), ((), ())),
                                preferred_element_type=jnp.float32)
            o = o + 8.0 * b2
            out_ref[:, pl.ds(2 * l2 * 64, 128)] = jnp.concatenate(
                [o[:Bb, :], o[Bb:, :]], axis=1)

    one_side(fA_ref, outA_ref)
    one_side(fB_ref, outB_ref)


def _chunk_major(x):
    return x.reshape(NCHUNK, NLANE, L).transpose(0, 2, 1).reshape(-1)


def _encode(fA, fB, w1c, b1c, w2, b2r):
    bh = fA.shape[0] * NLANE
    EBB = 128
    out_shape = jax.ShapeDtypeStruct((bh, L * 64), jnp.float32)
    return pl.pallas_call(
        _encode_body,
        grid=(bh // EBB,),
        in_specs=[
            pl.BlockSpec((8, FCW), lambda i: (i, 0)),
            pl.BlockSpec((8, FCW), lambda i: (i, 0)),
            pl.BlockSpec((64, 1), lambda i: (0, 0)),
            pl.BlockSpec((64, 1), lambda i: (0, 0)),
            pl.BlockSpec((64, 64), lambda i: (0, 0)),
            pl.BlockSpec((1, 64), lambda i: (0, 0)),
        ],
        out_specs=[
            pl.BlockSpec((EBB, L * 64), lambda i: (i, 0)),
            pl.BlockSpec((EBB, L * 64), lambda i: (i, 0)),
        ],
        out_shape=[out_shape, out_shape],
    )(fA, fB, w1c, b1c, w2, b2r)


@jax.jit
def _run(idsA, idsB, tA, tB, othA, othB, curt, w1c, b1c, w2, b2r):
    fA_f, fB_f = _sc_features(_chunk_major(idsA), _chunk_major(idsB),
                              _chunk_major(tA), _chunk_major(tB),
                              othA, othB, curt)
    outA, outB = _encode(fA_f.reshape(NCHUNK, FCW), fB_f.reshape(NCHUNK, FCW),
                         w1c, b1c, w2, b2r)
    return outA.reshape(B, L, 64), outB.reshape(B, L, 64)


def kernel(src_padded_nodes_neighbor_ids, dst_padded_nodes_neighbor_ids,
           src_node_ids, dst_node_ids, node_interact_times,
           src_padded_nodes_neighbor_times, dst_padded_nodes_neighbor_times,
           W1, b1, W2, b2):
    idsA = src_padded_nodes_neighbor_ids.astype(jnp.int32)
    idsB = dst_padded_nodes_neighbor_ids.astype(jnp.int32)
    othA = dst_node_ids.astype(jnp.int32)
    othB = src_node_ids.astype(jnp.int32)
    w1c = W1.reshape(1, 64).T
    b1c = b1.reshape(64, 1)
    b2r = b2.reshape(1, 64)
    return _run(idsA, idsB, src_padded_nodes_neighbor_times,
                dst_padded_nodes_neighbor_times, othA, othB,
                node_interact_times, w1c, b1c, W2, b2r)
